# baseline (speedup 1.0000x reference)
import jax
import jax.numpy as jnp
from jax.experimental import pallas as pl
from jax.experimental.pallas import tpu as pltpu

_NE = 1024
_D = 64
_HW = 1024
_B = 32
_NELEM = _B * _D * _HW


def _vq_body(x_ref, w_ref, q_ref, loss_ref, acc_ref):
    i = pl.program_id(0)
    n = pl.num_programs(0)
    xc = x_ref[0]
    w = w_ref[...]
    xt = jnp.transpose(xc)
    xsq = jnp.sum(xt * xt, axis=1, keepdims=True)
    wsq = jnp.sum(w * w, axis=1, keepdims=True)
    m = jax.lax.dot_general(xt, w, (((1,), (1,)), ((), ())),
                            preferred_element_type=jnp.float32)
    dist = (xsq + wsq.reshape(1, _NE)) - 2.0 * m + 1e-8
    dmin = jnp.min(dist, axis=1, keepdims=True)
    ids = jax.lax.broadcasted_iota(jnp.int32, dist.shape, 1)
    idx = jnp.min(jnp.where(dist == dmin, ids, _NE), axis=1, keepdims=True)
    oh = (ids == idx).astype(jnp.float32)
    qc = jax.lax.dot_general(w, oh, (((0,), (1,)), ((), ())),
                             preferred_element_type=jnp.float32)
    q_ref[0] = xc + (qc - xc)
    diff = qc - xc
    part = jnp.sum(jnp.sum(diff * diff, axis=1, keepdims=True),
                   axis=0, keepdims=True)

    @pl.when(i == 0)
    def _init():
        acc_ref[0, 0] = 0.0

    acc_ref[0, 0] += part[0, 0]

    @pl.when(i == n - 1)
    def _fini():
        mean = acc_ref[0, 0] * (1.0 / _NELEM)
        loss_ref[0, 0] = mean + 0.25 * mean


def kernel(inputs, W):
    shp = inputs.shape
    x3 = inputs.reshape(_B, _D, _HW)
    q, loss = pl.pallas_call(
        _vq_body,
        grid=(_B,),
        in_specs=[
            pl.BlockSpec((1, _D, _HW), lambda i: (i, 0, 0)),
            pl.BlockSpec((_NE, _D), lambda i: (0, 0)),
        ],
        out_specs=[
            pl.BlockSpec((1, _D, _HW), lambda i: (i, 0, 0)),
            pl.BlockSpec(memory_space=pltpu.SMEM),
        ],
        out_shape=[
            jax.ShapeDtypeStruct((_B, _D, _HW), jnp.float32),
            jax.ShapeDtypeStruct((1, 1), jnp.float32),
        ],
        scratch_shapes=[pltpu.SMEM((1, 1), jnp.float32)],
    )(x3, W)
    return q.reshape(shp), loss[0, 0]


# speedup vs baseline: 1.1381x; 1.1381x over previous
import jax
import jax.numpy as jnp
from jax.experimental import pallas as pl
from jax.experimental.pallas import tpu as pltpu

_NE = 1024
_D = 64
_HW = 1024
_B = 32
_NELEM = _B * _D * _HW


def _vq_body(x_ref, w_ref, q_ref, loss_ref, acc_ref):
    i = pl.program_id(0)
    n = pl.num_programs(0)
    xc = x_ref[0]
    w = w_ref[...]
    xsq = jnp.sum(xc * xc, axis=0, keepdims=True)
    wsq = jnp.sum(w * w, axis=1, keepdims=True)
    m = jax.lax.dot_general(w, xc, (((1,), (0,)), ((), ())),
                            preferred_element_type=jnp.float32)
    dist = (xsq + wsq) - 2.0 * m + 1e-8
    dmin = jnp.min(dist, axis=0, keepdims=True)
    ids = jax.lax.broadcasted_iota(jnp.int32, dist.shape, 0)
    idx = jnp.min(jnp.where(dist == dmin, ids, _NE), axis=0, keepdims=True)
    oh = (ids == idx).astype(jnp.float32)
    qc = jax.lax.dot_general(w, oh, (((0,), (0,)), ((), ())),
                             preferred_element_type=jnp.float32)
    q_ref[0] = xc + (qc - xc)
    diff = qc - xc
    part = jnp.sum(jnp.sum(diff * diff, axis=1, keepdims=True),
                   axis=0, keepdims=True)

    @pl.when(i == 0)
    def _init():
        acc_ref[0, 0] = 0.0

    acc_ref[0, 0] += part[0, 0]

    @pl.when(i == n - 1)
    def _fini():
        mean = acc_ref[0, 0] * (1.0 / _NELEM)
        loss_ref[0, 0] = mean + 0.25 * mean


def kernel(inputs, W):
    shp = inputs.shape
    x3 = inputs.reshape(_B, _D, _HW)
    q, loss = pl.pallas_call(
        _vq_body,
        grid=(_B,),
        in_specs=[
            pl.BlockSpec((1, _D, _HW), lambda i: (i, 0, 0)),
            pl.BlockSpec((_NE, _D), lambda i: (0, 0)),
        ],
        out_specs=[
            pl.BlockSpec((1, _D, _HW), lambda i: (i, 0, 0)),
            pl.BlockSpec(memory_space=pltpu.SMEM),
        ],
        out_shape=[
            jax.ShapeDtypeStruct((_B, _D, _HW), jnp.float32),
            jax.ShapeDtypeStruct((1, 1), jnp.float32),
        ],
        scratch_shapes=[pltpu.SMEM((1, 1), jnp.float32)],
    )(x3, W)
    return q.reshape(shp), loss[0, 0]


# speedup vs baseline: 1.3847x; 1.2167x over previous
import jax
import jax.numpy as jnp
from jax.experimental import pallas as pl
from jax.experimental.pallas import tpu as pltpu

_NE = 1024
_D = 64
_HW = 1024
_B = 32
_NELEM = _B * _D * _HW


def _vq_body(x_ref, w_ref, q_ref, loss_ref, acc_ref):
    i = pl.program_id(0)
    n = pl.num_programs(0)
    w = w_ref[...]
    wsq = jnp.sum(w * w, axis=1, keepdims=True)
    part = jnp.zeros((1, 1), jnp.float32)
    for j in range(x_ref.shape[0]):
        xc = x_ref[j]
        xsq = jnp.sum(xc * xc, axis=0, keepdims=True)
        m = jax.lax.dot_general(w, xc, (((1,), (0,)), ((), ())),
                                preferred_element_type=jnp.float32)
        dist = (xsq + wsq) - 2.0 * m + 1e-8
        idx = jnp.argmin(dist, axis=0).reshape(1, _HW)
        ids = jax.lax.broadcasted_iota(jnp.int32, dist.shape, 0)
        oh = (ids == idx).astype(jnp.float32)
        qc = jax.lax.dot_general(w, oh, (((0,), (0,)), ((), ())),
                                 preferred_element_type=jnp.float32)
        q_ref[j] = xc + (qc - xc)
        diff = qc - xc
        part = part + jnp.sum(jnp.sum(diff * diff, axis=1, keepdims=True),
                              axis=0, keepdims=True)

    @pl.when(i == 0)
    def _init():
        acc_ref[0, 0] = 0.0

    acc_ref[0, 0] += part[0, 0]

    @pl.when(i == n - 1)
    def _fini():
        mean = acc_ref[0, 0] * (1.0 / _NELEM)
        loss_ref[0, 0] = mean + 0.25 * mean


def kernel(inputs, W):
    shp = inputs.shape
    x3 = inputs.reshape(_B, _D, _HW)
    nb = 2
    q, loss = pl.pallas_call(
        _vq_body,
        grid=(_B // nb,),
        in_specs=[
            pl.BlockSpec((nb, _D, _HW), lambda i: (i, 0, 0)),
            pl.BlockSpec((_NE, _D), lambda i: (0, 0)),
        ],
        out_specs=[
            pl.BlockSpec((nb, _D, _HW), lambda i: (i, 0, 0)),
            pl.BlockSpec(memory_space=pltpu.SMEM),
        ],
        out_shape=[
            jax.ShapeDtypeStruct((_B, _D, _HW), jnp.float32),
            jax.ShapeDtypeStruct((1, 1), jnp.float32),
        ],
        scratch_shapes=[pltpu.SMEM((1, 1), jnp.float32)],
    )(x3, W)
    return q.reshape(shp), loss[0, 0]
